# baseline (speedup 1.0000x reference)
import dataclasses
import functools

import jax
import jax.numpy as jnp
from jax import lax
from jax.experimental import pallas as pl
from jax.experimental.pallas import tpu as pltpu
from jax.experimental.pallas import tpu_sc as plsc

NC = 2
NS = 16
L = 16
NW = NC * NS

B = 8192
D = 4096
DW = D // 2
BPW = B // NW
CH = 8
NCHUNK = BPW // CH


def _sc_embed(ids_flat, wte_words):
    mesh = plsc.VectorSubcoreMesh(core_axis_name="c", subcore_axis_name="s")
    cp = pltpu.CompilerParams()
    if "needs_layout_passes" in pltpu.CompilerParams.__dataclass_fields__:
        cp = dataclasses.replace(cp, needs_layout_passes=False)

    @functools.partial(
        pl.kernel,
        compiler_params=cp,
        out_type=jax.ShapeDtypeStruct((B * D,), jnp.float32),
        mesh=mesh,
        scratch_types=[
            pltpu.VMEM((BPW,), jnp.int32),
            pltpu.VMEM((CH, DW), jnp.uint32),
            pltpu.VMEM((CH * D,), jnp.float32),
            pltpu.SemaphoreType.DMA,
        ],
    )
    def k(ids_hbm, wte_hbm, out_hbm, idx_v, rows_v, fout_v, sem):
        wid = lax.axis_index("s") * NC + lax.axis_index("c")
        base = wid * BPW
        pltpu.sync_copy(ids_hbm.at[pl.ds(base, BPW)], idx_v)

        evens = 2 * lax.iota(jnp.int32, 16)
        odds = evens + 1
        himask = jnp.uint32(0xFFFF0000)

        @pl.loop(0, NCHUNK)
        def _chunk(c):
            pltpu.async_copy(
                wte_hbm.at[idx_v.at[pl.ds(c * CH, CH)]], rows_v, sem
            ).wait()

            for r in range(CH):
                @pl.loop(0, DW // L)
                def _col(j, r=r):
                    w = rows_v[r, pl.ds(j * L, L)]
                    lo = plsc.bitcast(w << 16, jnp.float32)
                    hi = plsc.bitcast(w & himask, jnp.float32)
                    fbase = r * D + j * 2 * L
                    plsc.store_scatter(fout_v, [fbase + evens], lo)
                    plsc.store_scatter(fout_v, [fbase + odds], hi)

            pltpu.sync_copy(
                fout_v, out_hbm.at[pl.ds((base + c * CH) * D, CH * D)]
            )

    return k(ids_flat, wte_words)


def kernel(input_ids, wte):
    ids_flat = input_ids.reshape(-1).astype(jnp.int32)
    wte_words = jax.lax.bitcast_convert_type(
        wte.reshape(wte.shape[0], DW, 2), jnp.uint32
    )
    out = _sc_embed(ids_flat, wte_words)
    return out.reshape(input_ids.shape[0], input_ids.shape[1], D)


# speedup vs baseline: 13.6384x; 13.6384x over previous
import dataclasses
import functools

import jax
import jax.numpy as jnp
from jax import lax
from jax.experimental import pallas as pl
from jax.experimental.pallas import tpu as pltpu
from jax.experimental.pallas import tpu_sc as plsc

NC = 2
NS = 16
NW = NC * NS

B = 8192
D = 4096
DW = D // 2
BPW = B // NW


def _sc_embed(ids_flat, wte):
    mesh = plsc.VectorSubcoreMesh(core_axis_name="c", subcore_axis_name="s")
    cp = pltpu.CompilerParams()
    if "needs_layout_passes" in pltpu.CompilerParams.__dataclass_fields__:
        cp = dataclasses.replace(cp, needs_layout_passes=False)

    @functools.partial(
        pl.kernel,
        compiler_params=cp,
        out_type=jax.ShapeDtypeStruct((B * D,), jnp.float32),
        mesh=mesh,
        scratch_types=[
            pltpu.VMEM((BPW,), jnp.int32),
            pltpu.VMEM((8, D), jnp.bfloat16),
            pltpu.VMEM((D,), jnp.float32),
            pltpu.SemaphoreType.DMA,
        ],
    )
    def k(ids_hbm, wte_hbm, out_hbm, idx_v, slab_v, fout_v, sem):
        wid = lax.axis_index("s") * NC + lax.axis_index("c")
        base = wid * BPW
        pltpu.sync_copy(ids_hbm.at[pl.ds(base, BPW)], idx_v)

        lanes = lax.iota(jnp.int32, 16)
        himask = jnp.uint32(0xFFFF0000)

        @pl.loop(0, BPW // 16)
        def _grp(g):
            tvec = idx_v[pl.ds(g * 16, 16)]
            trs = tvec >> 3
            rrs = tvec & 7

            for kk in range(16):
                tr = jnp.sum(jnp.where(lanes == kk, trs, 0))
                r = jnp.sum(jnp.where(lanes == kk, rrs, 0))
                pltpu.async_copy(wte_hbm.at[tr], slab_v, sem).wait()

                sh = jnp.where((r & 1) == 0, 16, 0).astype(jnp.uint32)

                @pl.loop(0, D // 16)
                def _col(j, r=r, sh=sh):
                    x = slab_v[r, pl.ds(j * 16, 32)]
                    w = plsc.bitcast(x, jnp.uint32)
                    y = plsc.bitcast((w << sh) & himask, jnp.float32)
                    plsc.store_scatter(fout_v, [j * 16 + lanes], y)

                t = g * 16 + kk
                pltpu.sync_copy(fout_v, out_hbm.at[pl.ds((base + t) * D, D)])

    return k(ids_flat, wte)


def kernel(input_ids, wte):
    ids_flat = input_ids.reshape(-1).astype(jnp.int32)
    wte3 = wte.reshape(wte.shape[0] // 8, 8, D)
    out = _sc_embed(ids_flat, wte3)
    return out.reshape(input_ids.shape[0], input_ids.shape[1], D)


# speedup vs baseline: 22.9168x; 1.6803x over previous
import dataclasses
import functools

import jax
import jax.numpy as jnp
from jax import lax
from jax.experimental import pallas as pl
from jax.experimental.pallas import tpu as pltpu
from jax.experimental.pallas import tpu_sc as plsc

NC = 2
NS = 16
NW = NC * NS

B = 8192
D = 4096
BPW = B // NW
NBUF = 4
NFB = 2


def _sc_embed(ids_flat, wte3):
    mesh = plsc.VectorSubcoreMesh(core_axis_name="c", subcore_axis_name="s")
    cp = pltpu.CompilerParams()
    if "needs_layout_passes" in pltpu.CompilerParams.__dataclass_fields__:
        cp = dataclasses.replace(cp, needs_layout_passes=False)

    @functools.partial(
        pl.kernel,
        compiler_params=cp,
        out_type=jax.ShapeDtypeStruct((B * D,), jnp.float32),
        mesh=mesh,
        scratch_types=[
            pltpu.VMEM((BPW,), jnp.int32),
            pltpu.VMEM((8, D), jnp.bfloat16),
            pltpu.VMEM((8, D), jnp.bfloat16),
            pltpu.VMEM((8, D), jnp.bfloat16),
            pltpu.VMEM((8, D), jnp.bfloat16),
            pltpu.VMEM((D,), jnp.float32),
            pltpu.VMEM((D,), jnp.float32),
            pltpu.SemaphoreType.DMA,
            pltpu.SemaphoreType.DMA,
            pltpu.SemaphoreType.DMA,
            pltpu.SemaphoreType.DMA,
            pltpu.SemaphoreType.DMA,
            pltpu.SemaphoreType.DMA,
        ],
    )
    def k(ids_hbm, wte_hbm, out_hbm, idx_v, sl0, sl1, sl2, sl3, f0, f1,
          s0, s1, s2, s3, w0, w1):
        slabs = (sl0, sl1, sl2, sl3)
        fouts = (f0, f1)
        gsem = (s0, s1, s2, s3)
        wsem = (w0, w1)
        wid = lax.axis_index("s") * NC + lax.axis_index("c")
        base = wid * BPW
        pltpu.sync_copy(ids_hbm.at[pl.ds(base, BPW)], idx_v)

        lanes = lax.iota(jnp.int32, 16)
        himask = jnp.uint32(0xFFFF0000)

        def row_of(u):
            tv = idx_v[pl.ds((u >> 4) * 16, 16)]
            return jnp.sum(jnp.where(lanes == (u & 15), tv, 0))

        for b in range(NBUF):
            pltpu.make_async_copy(
                wte_hbm.at[row_of(b) >> 3], slabs[b], gsem[b]
            ).start()

        @pl.loop(0, BPW // NBUF)
        def _chunk(cc):
            for b in range(NBUF):
                t = cc * NBUF + b
                pltpu.make_async_copy(
                    wte_hbm.at[0], slabs[b], gsem[b]
                ).wait()

                row = row_of(t)
                r = row & 7
                sh = jnp.where((r & 1) == 0, 16, 0).astype(jnp.uint32)
                fb = b & 1

                @pl.when(t >= NFB)
                def _():
                    pltpu.make_async_copy(
                        fouts[fb], out_hbm.at[pl.ds(0, D)], wsem[fb]
                    ).wait()

                @pl.loop(0, D // 64)
                def _col(jj, r=r, sh=sh, fb=fb, b=b):
                    for u in range(4):
                        j = jj * 4 + u
                        x = slabs[b][r, pl.ds(j * 16, 32)]
                        w = plsc.bitcast(x, jnp.uint32)
                        y = plsc.bitcast((w << sh) & himask, jnp.float32)
                        plsc.store_scatter(fouts[fb], [j * 16 + lanes], y)

                pltpu.make_async_copy(
                    fouts[fb], out_hbm.at[pl.ds((base + t) * D, D)],
                    wsem[fb],
                ).start()

                @pl.when(t + NBUF < BPW)
                def _():
                    pltpu.make_async_copy(
                        wte_hbm.at[row_of(t + NBUF) >> 3], slabs[b],
                        gsem[b],
                    ).start()

        for fb in range(NFB):
            pltpu.make_async_copy(
                fouts[fb], out_hbm.at[pl.ds(0, D)], wsem[fb]
            ).wait()

    return k(ids_flat, wte3)


def kernel(input_ids, wte):
    ids_flat = input_ids.reshape(-1).astype(jnp.int32)
    wte3 = wte.reshape(wte.shape[0] // 8, 8, D)
    out = _sc_embed(ids_flat, wte3)
    return out.reshape(input_ids.shape[0], input_ids.shape[1], D)


# speedup vs baseline: 31.4012x; 1.3702x over previous
import dataclasses
import functools

import jax
import jax.numpy as jnp
from jax import lax
from jax.experimental import pallas as pl
from jax.experimental.pallas import tpu as pltpu
from jax.experimental.pallas import tpu_sc as plsc

NC = 2
NS = 16
NW = NC * NS

B = 8192
D = 4096
BPW = B // NW
NBUF = 4
NFB = 2


def _sc_embed(ids_flat, wte3):
    mesh = plsc.VectorSubcoreMesh(core_axis_name="c", subcore_axis_name="s")
    cp = pltpu.CompilerParams()
    if "needs_layout_passes" in pltpu.CompilerParams.__dataclass_fields__:
        cp = dataclasses.replace(cp, needs_layout_passes=False)

    @functools.partial(
        pl.kernel,
        compiler_params=cp,
        out_type=jax.ShapeDtypeStruct((B * D,), jnp.float32),
        mesh=mesh,
        scratch_types=[
            pltpu.VMEM((BPW,), jnp.int32),
            pltpu.VMEM((8, D), jnp.bfloat16),
            pltpu.VMEM((8, D), jnp.bfloat16),
            pltpu.VMEM((8, D), jnp.bfloat16),
            pltpu.VMEM((8, D), jnp.bfloat16),
            pltpu.VMEM((D,), jnp.float32),
            pltpu.VMEM((D,), jnp.float32),
            pltpu.SemaphoreType.DMA,
            pltpu.SemaphoreType.DMA,
            pltpu.SemaphoreType.DMA,
            pltpu.SemaphoreType.DMA,
            pltpu.SemaphoreType.DMA,
            pltpu.SemaphoreType.DMA,
        ],
    )
    def k(ids_hbm, wte_hbm, out_hbm, idx_v, sl0, sl1, sl2, sl3, f0, f1,
          s0, s1, s2, s3, w0, w1):
        slabs = (sl0, sl1, sl2, sl3)
        fouts = (f0, f1)
        gsem = (s0, s1, s2, s3)
        wsem = (w0, w1)
        wid = lax.axis_index("s") * NC + lax.axis_index("c")
        base = wid * BPW
        pltpu.sync_copy(ids_hbm.at[pl.ds(base, BPW)], idx_v)

        lanes = lax.iota(jnp.int32, 16)
        himask = jnp.uint32(0xFFFF0000)

        def row_of(u):
            tv = idx_v[pl.ds((u >> 4) * 16, 16)]
            return jnp.sum(jnp.where(lanes == (u & 15), tv, 0))

        for b in range(NBUF):
            pltpu.make_async_copy(
                wte_hbm.at[row_of(b) >> 3], slabs[b], gsem[b]
            ).start()

        @pl.loop(0, BPW // NBUF)
        def _chunk(cc):
            for b in range(NBUF):
                t = cc * NBUF + b
                pltpu.make_async_copy(
                    wte_hbm.at[0], slabs[b], gsem[b]
                ).wait()

                row = row_of(t)
                r = row & 7
                sh = jnp.where((r & 1) == 0, 16, 0).astype(jnp.uint32)
                fb = b & 1

                @pl.when(t >= NFB)
                def _():
                    pltpu.make_async_copy(
                        fouts[fb], out_hbm.at[pl.ds(0, D)], wsem[fb]
                    ).wait()

                @plsc.parallel_loop(0, D // 16, unroll=8)
                def _col(j, r=r, sh=sh, fb=fb, b=b):
                    x = slabs[b][r, pl.ds(j * 16, 32)]
                    w = plsc.bitcast(x, jnp.uint32)
                    y = plsc.bitcast((w << sh) & himask, jnp.float32)
                    plsc.store_scatter(fouts[fb], [j * 16 + lanes], y)

                pltpu.make_async_copy(
                    fouts[fb], out_hbm.at[pl.ds((base + t) * D, D)],
                    wsem[fb],
                ).start()

                @pl.when(t + NBUF < BPW)
                def _():
                    pltpu.make_async_copy(
                        wte_hbm.at[row_of(t + NBUF) >> 3], slabs[b],
                        gsem[b],
                    ).start()

        for fb in range(NFB):
            pltpu.make_async_copy(
                fouts[fb], out_hbm.at[pl.ds(0, D)], wsem[fb]
            ).wait()

    return k(ids_flat, wte3)


def kernel(input_ids, wte):
    ids_flat = input_ids.reshape(-1).astype(jnp.int32)
    wte3 = wte.reshape(wte.shape[0] // 8, 8, D)
    out = _sc_embed(ids_flat, wte3)
    return out.reshape(input_ids.shape[0], input_ids.shape[1], D)
